# Initial kernel scaffold; baseline (speedup 1.0000x reference)
import jax
import jax.numpy as jnp
from jax.experimental import pallas as pl


def kernel(x, edge_index, edge_attr, W1, b1, W2, b2, W3, b3):
    raise NotImplementedError("write your pallas kernel here")


# speedup vs baseline: 12.7868x; 12.7868x over previous
import functools

import jax
import jax.numpy as jnp
from jax import lax
from jax.experimental import pallas as pl
from jax.experimental.pallas import tpu as pltpu
from jax.experimental.pallas import tpu_sc as plsc

N = 10000
D = 128
NC = 2
NS = 16
NW = NC * NS
N_PAD = 10112
RPT = N_PAD // NS
CH = 128
NCHUNK = 80
EPT = NCHUNK * CH
E_PAD = NW * EPT

BM = 400
GRID_M = N // BM


def _sc_deg_body(dst_hbm, ones_hbm, zeros_hbm, out_hbm, didx, ones_v, acc_sh):
    c = lax.axis_index("c")
    s = lax.axis_index("s")
    wid = c * NS + s
    pltpu.sync_copy(ones_hbm, ones_v)
    pltpu.sync_copy(zeros_hbm.at[pl.ds(s * RPT, RPT)],
                    acc_sh.at[pl.ds(s * RPT, RPT)])
    plsc.subcore_barrier()
    base = wid * EPT

    def body(j, carry):
        pltpu.sync_copy(dst_hbm.at[pl.ds(base + j * CH, CH)], didx)
        pltpu.sync_copy(ones_v, acc_sh.at[didx], add=True)
        return carry

    lax.fori_loop(0, NCHUNK, body, 0)
    plsc.subcore_barrier()
    pltpu.sync_copy(acc_sh.at[pl.ds(s * RPT, RPT)],
                    out_hbm.at[pl.ds(c * N_PAD + s * RPT, RPT)])


_sc_deg = pl.kernel(
    _sc_deg_body,
    out_type=jax.ShapeDtypeStruct((NC * N_PAD, 16), jnp.float32),
    mesh=plsc.VectorSubcoreMesh(core_axis_name="c", subcore_axis_name="s"),
    scratch_types=[
        pltpu.VMEM((CH,), jnp.int32),
        pltpu.VMEM((CH, 16), jnp.float32),
        pltpu.VMEM_SHARED((N_PAD, 16), jnp.float32),
    ],
    compiler_params=pltpu.CompilerParams(use_tc_tiling_on_sc=False),
)


def _sc_agg_body(y_hbm, src_hbm, dst_hbm, zeros_hbm, out_hbm,
                 sidx, didx, rows, acc_sh):
    c = lax.axis_index("c")
    s = lax.axis_index("s")
    wid = c * NS + s
    pltpu.sync_copy(zeros_hbm.at[pl.ds(s * RPT, RPT)],
                    acc_sh.at[pl.ds(s * RPT, RPT)])
    plsc.subcore_barrier()
    base = wid * EPT

    def body(j, carry):
        off = base + j * CH
        pltpu.sync_copy(src_hbm.at[pl.ds(off, CH)], sidx)
        pltpu.sync_copy(dst_hbm.at[pl.ds(off, CH)], didx)
        pltpu.sync_copy(y_hbm.at[sidx], rows)
        pltpu.sync_copy(rows, acc_sh.at[didx], add=True)
        return carry

    lax.fori_loop(0, NCHUNK, body, 0)
    plsc.subcore_barrier()
    pltpu.sync_copy(acc_sh.at[pl.ds(s * RPT, RPT)],
                    out_hbm.at[pl.ds(c * N_PAD + s * RPT, RPT)])


_sc_agg = pl.kernel(
    _sc_agg_body,
    out_type=jax.ShapeDtypeStruct((NC * N_PAD, D), jnp.float32),
    mesh=plsc.VectorSubcoreMesh(core_axis_name="c", subcore_axis_name="s"),
    scratch_types=[
        pltpu.VMEM((CH,), jnp.int32),
        pltpu.VMEM((CH,), jnp.int32),
        pltpu.VMEM((CH, D), jnp.float32),
        pltpu.VMEM_SHARED((N_PAD, D), jnp.float32),
    ],
)


def _dis_block(d0_ref, d1_ref):
    return lax.rsqrt(d0_ref[...][:, 0:1] + d1_ref[...][:, 0:1] + 1.0)


def _tc_l1_body(x_ref, w_ref, d0_ref, d1_ref, y_ref):
    dis = _dis_block(d0_ref, d1_ref)
    y_ref[...] = dis * jnp.dot(x_ref[...], w_ref[...],
                               preferred_element_type=jnp.float32)


def _tc_mid_body(a0_ref, a1_ref, yp_ref, d0_ref, d1_ref, b_ref, w_ref, y_ref):
    dis = _dis_block(d0_ref, d1_ref)
    pre = dis * (a0_ref[...] + a1_ref[...] + yp_ref[...]) + b_ref[...]
    h = jax.nn.sigmoid(pre)
    y_ref[...] = dis * jnp.dot(h, w_ref[...],
                               preferred_element_type=jnp.float32)


def _tc_fin_body(a0_ref, a1_ref, yp_ref, d0_ref, d1_ref, b_ref, out_ref):
    dis = _dis_block(d0_ref, d1_ref)
    out_ref[...] = dis * (a0_ref[...] + a1_ref[...] + yp_ref[...]) + b_ref[...]


_row_spec = pl.BlockSpec((BM, D), lambda m: (m, 0))
_deg_spec = pl.BlockSpec((BM, 16), lambda m: (m, 0))
_w_spec = pl.BlockSpec((D, D), lambda m: (0, 0))
_b_spec = pl.BlockSpec((1, D), lambda m: (0, 0))
_out_sds = jax.ShapeDtypeStruct((N, D), jnp.float32)

_tc_l1 = pl.pallas_call(
    _tc_l1_body,
    grid=(GRID_M,),
    in_specs=[_row_spec, _w_spec, _deg_spec, _deg_spec],
    out_specs=_row_spec,
    out_shape=_out_sds,
)

_tc_mid = pl.pallas_call(
    _tc_mid_body,
    grid=(GRID_M,),
    in_specs=[_row_spec, _row_spec, _row_spec, _deg_spec, _deg_spec,
              _b_spec, _w_spec],
    out_specs=_row_spec,
    out_shape=_out_sds,
)

_tc_fin = pl.pallas_call(
    _tc_fin_body,
    grid=(GRID_M,),
    in_specs=[_row_spec, _row_spec, _row_spec, _deg_spec, _deg_spec, _b_spec],
    out_specs=_row_spec,
    out_shape=_out_sds,
)


def kernel(x, edge_index, edge_attr, W1, b1, W2, b2, W3, b3):
    src = edge_index[0].astype(jnp.int32)
    dst = edge_index[1].astype(jnp.int32)
    pad = E_PAD - src.shape[0]
    ar = jnp.arange(pad, dtype=jnp.int32)
    src_p = jnp.concatenate([src, ar % N])
    dst_p = jnp.concatenate([dst, N + (ar % (N_PAD - N))])

    ones16 = jnp.ones((CH, 16), jnp.float32)
    zeros16 = jnp.zeros((N_PAD, 16), jnp.float32)
    zerosD = jnp.zeros((N_PAD, D), jnp.float32)

    degp = _sc_deg(dst_p, ones16, zeros16)
    deg0 = degp[:N_PAD]
    deg1 = degp[N_PAD:]

    b1r = b1.reshape(1, D)
    b2r = b2.reshape(1, D)
    b3r = b3.reshape(1, D)

    y1 = _tc_l1(x, W1, deg0, deg1)
    acc = _sc_agg(y1, src_p, dst_p, zerosD)
    y2 = _tc_mid(acc[:N], acc[N_PAD:N_PAD + N], y1, deg0, deg1, b1r, W2)
    acc = _sc_agg(y2, src_p, dst_p, zerosD)
    y3 = _tc_mid(acc[:N], acc[N_PAD:N_PAD + N], y2, deg0, deg1, b2r, W3)
    acc = _sc_agg(y3, src_p, dst_p, zerosD)
    return _tc_fin(acc[:N], acc[N_PAD:N_PAD + N], y3, deg0, deg1, b3r)


# speedup vs baseline: 23.2812x; 1.8207x over previous
import functools

import jax
import jax.numpy as jnp
from jax import lax
from jax.experimental import pallas as pl
from jax.experimental.pallas import tpu as pltpu
from jax.experimental.pallas import tpu_sc as plsc

N = 10000
D = 128
NC = 2
NS = 16
NW = NC * NS
N_PAD = 10112
RPT = N_PAD // NS
CH = 128
NCHUNK = 80
EPT = NCHUNK * CH
E_PER_TILE = 10000
G = 2
NGRP = NCHUNK // G

BM = 400
GRID_M = N // BM


def _sc_deg_body(dst_hbm, ones_hbm, zeros_hbm, out_hbm,
                 di0, di1, ones_v, acc_sh, id0, id1, s0, s1):
    c = lax.axis_index("c")
    s = lax.axis_index("s")
    wid = c * NS + s
    dib = (di0, di1)
    idsem = (id0, id1)
    ss = (s0, s1)
    tile_dst = dst_hbm.at[wid]
    pltpu.sync_copy(ones_hbm, ones_v)
    pltpu.sync_copy(zeros_hbm.at[pl.ds(s * RPT, RPT)],
                    acc_sh.at[pl.ds(s * RPT, RPT)])
    pltpu.sync_copy(tile_dst.at[pl.ds(0, G)], dib[0])
    pltpu.async_copy(tile_dst.at[pl.ds(G, G)], dib[1], idsem[1])
    plsc.subcore_barrier()

    def body(jp, carry):
        for k in range(2):
            jg = jp * 2 + k
            cur, nxt = k, 1 - k

            @pl.when(jg + 1 < NGRP)
            def _():
                pltpu.make_async_copy(tile_dst.at[pl.ds(0, G)],
                                      dib[nxt], idsem[nxt]).wait()

            for b in range(G):
                pltpu.async_copy(ones_v, acc_sh.at[dib[cur].at[b]],
                                 ss[b], add=True)
            for b in range(G):
                pltpu.make_async_copy(ones_v, acc_sh.at[dib[cur].at[0]],
                                      ss[b]).wait()

            @pl.when(jg + 2 < NGRP)
            def _(jg=jg, cur=cur):
                pltpu.async_copy(tile_dst.at[pl.ds((jg + 2) * G, G)],
                                 dib[cur], idsem[cur])
        return carry

    lax.fori_loop(0, NGRP // 2, body, 0)
    plsc.subcore_barrier()
    pltpu.sync_copy(acc_sh.at[pl.ds(s * RPT, RPT)],
                    out_hbm.at[pl.ds(c * N_PAD + s * RPT, RPT)])


_sc_deg = pl.kernel(
    _sc_deg_body,
    out_type=jax.ShapeDtypeStruct((NC * N_PAD, 16), jnp.float32),
    mesh=plsc.VectorSubcoreMesh(core_axis_name="c", subcore_axis_name="s"),
    scratch_types=[
        pltpu.VMEM((G, CH), jnp.int32),
        pltpu.VMEM((G, CH), jnp.int32),
        pltpu.VMEM((CH, 16), jnp.float32),
        pltpu.VMEM_SHARED((N_PAD, 16), jnp.float32),
        pltpu.SemaphoreType.DMA,
        pltpu.SemaphoreType.DMA,
        pltpu.SemaphoreType.DMA,
        pltpu.SemaphoreType.DMA,
    ],
    compiler_params=pltpu.CompilerParams(use_tc_tiling_on_sc=False),
)


def _sc_agg_body(y_hbm, src_hbm, dst_hbm, zeros_hbm, out_hbm,
                 si0, si1, di0, di1, r0, r1, acc_sh,
                 is0, is1, id0, id1, g0, g1, s0, s1):
    c = lax.axis_index("c")
    s = lax.axis_index("s")
    wid = c * NS + s
    sib = (si0, si1)
    dib = (di0, di1)
    isem = (is0, is1)
    idsem = (id0, id1)
    rows = (r0, r1)
    gs = (g0, g1)
    ss = (s0, s1)
    tile_src = src_hbm.at[wid]
    tile_dst = dst_hbm.at[wid]
    pltpu.sync_copy(zeros_hbm.at[pl.ds(s * RPT, RPT)],
                    acc_sh.at[pl.ds(s * RPT, RPT)])
    pltpu.sync_copy(tile_src.at[pl.ds(0, G)], sib[0])
    pltpu.sync_copy(tile_dst.at[pl.ds(0, G)], dib[0])
    pltpu.async_copy(tile_src.at[pl.ds(G, G)], sib[1], isem[1])
    pltpu.async_copy(tile_dst.at[pl.ds(G, G)], dib[1], idsem[1])
    plsc.subcore_barrier()

    for b in range(G):
        pltpu.async_copy(y_hbm.at[sib[0].at[b]], rows[b], gs[b])

    def body(jp, carry):
        for k in range(2):
            jg = jp * 2 + k
            cur, nxt = k, 1 - k

            @pl.when(jg + 1 < NGRP)
            def _():
                pltpu.make_async_copy(tile_src.at[pl.ds(0, G)],
                                      sib[nxt], isem[nxt]).wait()
                pltpu.make_async_copy(tile_dst.at[pl.ds(0, G)],
                                      dib[nxt], idsem[nxt]).wait()

            for b in range(G):
                pltpu.make_async_copy(y_hbm.at[sib[cur].at[b]],
                                      rows[b], gs[b]).wait()
                pltpu.async_copy(rows[b], acc_sh.at[dib[cur].at[b]],
                                 ss[b], add=True).wait()

                @pl.when(jg + 1 < NGRP)
                def _(b=b):
                    pltpu.async_copy(y_hbm.at[sib[nxt].at[b]], rows[b], gs[b])

            @pl.when(jg + 2 < NGRP)
            def _(jg=jg, cur=cur):
                pltpu.async_copy(tile_src.at[pl.ds((jg + 2) * G, G)],
                                 sib[cur], isem[cur])
                pltpu.async_copy(tile_dst.at[pl.ds((jg + 2) * G, G)],
                                 dib[cur], idsem[cur])
        return carry

    lax.fori_loop(0, NGRP // 2, body, 0)
    plsc.subcore_barrier()
    pltpu.sync_copy(acc_sh.at[pl.ds(s * RPT, RPT)],
                    out_hbm.at[pl.ds(c * N_PAD + s * RPT, RPT)])


_sc_agg = pl.kernel(
    _sc_agg_body,
    out_type=jax.ShapeDtypeStruct((NC * N_PAD, D), jnp.float32),
    mesh=plsc.VectorSubcoreMesh(core_axis_name="c", subcore_axis_name="s"),
    scratch_types=[
        pltpu.VMEM((G, CH), jnp.int32),
        pltpu.VMEM((G, CH), jnp.int32),
        pltpu.VMEM((G, CH), jnp.int32),
        pltpu.VMEM((G, CH), jnp.int32),
        pltpu.VMEM((CH, D), jnp.float32),
        pltpu.VMEM((CH, D), jnp.float32),
        pltpu.VMEM_SHARED((N_PAD, D), jnp.float32),
        pltpu.SemaphoreType.DMA,
        pltpu.SemaphoreType.DMA,
        pltpu.SemaphoreType.DMA,
        pltpu.SemaphoreType.DMA,
        pltpu.SemaphoreType.DMA,
        pltpu.SemaphoreType.DMA,
        pltpu.SemaphoreType.DMA,
        pltpu.SemaphoreType.DMA,
    ],
)


def _dis_block(d0_ref, d1_ref):
    return lax.rsqrt(d0_ref[...][:, 0:1] + d1_ref[...][:, 0:1] + 1.0)


def _tc_l1_body(x_ref, w_ref, d0_ref, d1_ref, y_ref):
    dis = _dis_block(d0_ref, d1_ref)
    y_ref[...] = dis * jnp.dot(x_ref[...], w_ref[...],
                               preferred_element_type=jnp.float32)


def _tc_mid_body(a0_ref, a1_ref, yp_ref, d0_ref, d1_ref, b_ref, w_ref, y_ref):
    dis = _dis_block(d0_ref, d1_ref)
    pre = dis * (a0_ref[...] + a1_ref[...] + yp_ref[...]) + b_ref[...]
    h = jax.nn.sigmoid(pre)
    y_ref[...] = dis * jnp.dot(h, w_ref[...],
                               preferred_element_type=jnp.float32)


def _tc_fin_body(a0_ref, a1_ref, yp_ref, d0_ref, d1_ref, b_ref, out_ref):
    dis = _dis_block(d0_ref, d1_ref)
    out_ref[...] = dis * (a0_ref[...] + a1_ref[...] + yp_ref[...]) + b_ref[...]


_row_spec = pl.BlockSpec((BM, D), lambda m: (m, 0))
_deg_spec = pl.BlockSpec((BM, 16), lambda m: (m, 0))
_w_spec = pl.BlockSpec((D, D), lambda m: (0, 0))
_b_spec = pl.BlockSpec((1, D), lambda m: (0, 0))
_out_sds = jax.ShapeDtypeStruct((N, D), jnp.float32)

_tc_l1 = pl.pallas_call(
    _tc_l1_body,
    grid=(GRID_M,),
    in_specs=[_row_spec, _w_spec, _deg_spec, _deg_spec],
    out_specs=_row_spec,
    out_shape=_out_sds,
)

_tc_mid = pl.pallas_call(
    _tc_mid_body,
    grid=(GRID_M,),
    in_specs=[_row_spec, _row_spec, _row_spec, _deg_spec, _deg_spec,
              _b_spec, _w_spec],
    out_specs=_row_spec,
    out_shape=_out_sds,
)

_tc_fin = pl.pallas_call(
    _tc_fin_body,
    grid=(GRID_M,),
    in_specs=[_row_spec, _row_spec, _row_spec, _deg_spec, _deg_spec, _b_spec],
    out_specs=_row_spec,
    out_shape=_out_sds,
)


def kernel(x, edge_index, edge_attr, W1, b1, W2, b2, W3, b3):
    src = edge_index[0].astype(jnp.int32).reshape(NW, E_PER_TILE)
    dst = edge_index[1].astype(jnp.int32).reshape(NW, E_PER_TILE)
    padw = EPT - E_PER_TILE
    ar = jnp.arange(padw, dtype=jnp.int32)
    pad_src = jnp.broadcast_to(ar % N, (NW, padw))
    pad_dst = jnp.broadcast_to(N + (ar % (N_PAD - N)), (NW, padw))
    src_p = jnp.concatenate([src, pad_src], axis=1).reshape(NW, NCHUNK, CH)
    dst_p = jnp.concatenate([dst, pad_dst], axis=1).reshape(NW, NCHUNK, CH)

    ones16 = jnp.ones((CH, 16), jnp.float32)
    zeros16 = jnp.zeros((N_PAD, 16), jnp.float32)
    zerosD = jnp.zeros((N_PAD, D), jnp.float32)

    degp = _sc_deg(dst_p, ones16, zeros16)
    deg0 = degp[:N_PAD]
    deg1 = degp[N_PAD:]

    b1r = b1.reshape(1, D)
    b2r = b2.reshape(1, D)
    b3r = b3.reshape(1, D)

    y1 = _tc_l1(x, W1, deg0, deg1)
    acc = _sc_agg(y1, src_p, dst_p, zerosD)
    y2 = _tc_mid(acc[:N], acc[N_PAD:N_PAD + N], y1, deg0, deg1, b1r, W2)
    acc = _sc_agg(y2, src_p, dst_p, zerosD)
    y3 = _tc_mid(acc[:N], acc[N_PAD:N_PAD + N], y2, deg0, deg1, b2r, W3)
    acc = _sc_agg(y3, src_p, dst_p, zerosD)
    return _tc_fin(acc[:N], acc[N_PAD:N_PAD + N], y3, deg0, deg1, b3r)


# speedup vs baseline: 25.3417x; 1.0885x over previous
import functools

import jax
import jax.numpy as jnp
from jax import lax
from jax.experimental import pallas as pl
from jax.experimental.pallas import tpu as pltpu
from jax.experimental.pallas import tpu_sc as plsc

N = 10000
D = 128
HD = D // 2
NC = 2
NS = 16
NW = NC * NS
N_PAD = 10112
RPT = N_PAD // NS
CH = 128
E_PAD = 327680
NCHUNK_D = 80
NCHUNK_A = 160
G = 4
NGRP_D = NCHUNK_D // G
NGRP_A = NCHUNK_A // G

BM = 632
GRID_M = N_PAD // BM


def _sc_deg_body(dst_hbm, ones_hbm, zeros_hbm, out_hbm,
                 di0, di1, ones_v, acc_sh, id0, id1, s0, s1, s2, s3):
    c = lax.axis_index("c")
    s = lax.axis_index("s")
    wid = c * NS + s
    dib = (di0, di1)
    idsem = (id0, id1)
    ss = (s0, s1, s2, s3)
    tile_dst = dst_hbm.at[wid]
    pltpu.sync_copy(ones_hbm, ones_v)
    pltpu.sync_copy(zeros_hbm.at[pl.ds(s * RPT, RPT)],
                    acc_sh.at[pl.ds(s * RPT, RPT)])
    pltpu.sync_copy(tile_dst.at[pl.ds(0, G)], dib[0])
    pltpu.async_copy(tile_dst.at[pl.ds(G, G)], dib[1], idsem[1])
    plsc.subcore_barrier()

    def body(jp, carry):
        for k in range(2):
            jg = jp * 2 + k
            cur, nxt = k, 1 - k

            @pl.when(jg + 1 < NGRP_D)
            def _():
                pltpu.make_async_copy(tile_dst.at[pl.ds(0, G)],
                                      dib[nxt], idsem[nxt]).wait()

            for b in range(G):
                pltpu.async_copy(ones_v, acc_sh.at[dib[cur].at[b]],
                                 ss[b], add=True)
            for b in range(G):
                pltpu.make_async_copy(ones_v, acc_sh.at[dib[cur].at[0]],
                                      ss[b]).wait()

            @pl.when(jg + 2 < NGRP_D)
            def _(jg=jg, cur=cur):
                pltpu.async_copy(tile_dst.at[pl.ds((jg + 2) * G, G)],
                                 dib[cur], idsem[cur])
        return carry

    lax.fori_loop(0, NGRP_D // 2, body, 0)
    plsc.subcore_barrier()
    pltpu.sync_copy(acc_sh.at[pl.ds(s * RPT, RPT)],
                    out_hbm.at[pl.ds(c * N_PAD + s * RPT, RPT)])


_sc_deg = pl.kernel(
    _sc_deg_body,
    out_type=jax.ShapeDtypeStruct((NC * N_PAD, 16), jnp.float32),
    mesh=plsc.VectorSubcoreMesh(core_axis_name="c", subcore_axis_name="s"),
    scratch_types=[
        pltpu.VMEM((G, CH), jnp.int32),
        pltpu.VMEM((G, CH), jnp.int32),
        pltpu.VMEM((CH, 16), jnp.float32),
        pltpu.VMEM_SHARED((N_PAD, 16), jnp.float32),
        pltpu.SemaphoreType.DMA,
        pltpu.SemaphoreType.DMA,
        pltpu.SemaphoreType.DMA,
        pltpu.SemaphoreType.DMA,
        pltpu.SemaphoreType.DMA,
        pltpu.SemaphoreType.DMA,
    ],
    compiler_params=pltpu.CompilerParams(use_tc_tiling_on_sc=False),
)


def _sc_agg_body(y_hbm, src_hbm, dst_hbm, zeros_hbm, out_hbm,
                 si0, si1, di0, di1,
                 ra0, ra1, ra2, ra3, rb0, rb1, rb2, rb3, acc_sh,
                 is0, is1, id0, id1,
                 ga0, ga1, ga2, ga3, gb0, gb1, gb2, gb3,
                 sa0, sa1, sa2, sa3, sb0, sb1, sb2, sb3):
    c = lax.axis_index("c")
    s = lax.axis_index("s")
    sib = (si0, si1)
    dib = (di0, di1)
    isem = (is0, is1)
    idsem = (id0, id1)
    rows = ((ra0, ra1, ra2, ra3), (rb0, rb1, rb2, rb3))
    gsem = ((ga0, ga1, ga2, ga3), (gb0, gb1, gb2, gb3))
    ssem = ((sa0, sa1, sa2, sa3), (sb0, sb1, sb2, sb3))
    tile_src = src_hbm.at[c].at[s]
    tile_dst = dst_hbm.at[s]
    pltpu.sync_copy(zeros_hbm.at[pl.ds(s * RPT, RPT)],
                    acc_sh.at[pl.ds(s * RPT, RPT)])
    pltpu.sync_copy(tile_src.at[pl.ds(0, G)], sib[0])
    pltpu.sync_copy(tile_dst.at[pl.ds(0, G)], dib[0])
    pltpu.async_copy(tile_src.at[pl.ds(G, G)], sib[1], isem[1])
    plsc.subcore_barrier()

    for b in range(G):
        pltpu.async_copy(y_hbm.at[sib[0].at[b]], rows[0][b], gsem[0][b])

    def body(jp, carry):
        for k in range(2):
            jg = jp * 2 + k
            cur, nxt = k, 1 - k

            @pl.when(jg >= 1)
            def _():
                pltpu.make_async_copy(tile_dst.at[pl.ds(0, G)],
                                      dib[cur], idsem[cur]).wait()

            @pl.when(jg + 1 < NGRP_A)
            def _():
                pltpu.make_async_copy(tile_src.at[pl.ds(0, G)],
                                      sib[nxt], isem[nxt]).wait()

            for b in range(G):
                pltpu.make_async_copy(y_hbm.at[sib[cur].at[b]],
                                      rows[k][b], gsem[k][b]).wait()
                pltpu.async_copy(rows[k][b], acc_sh.at[dib[cur].at[b]],
                                 ssem[k][b], add=True)

                @pl.when(jg >= 1)
                def _(b=b):
                    pltpu.make_async_copy(rows[nxt][b],
                                          acc_sh.at[dib[cur].at[0]],
                                          ssem[nxt][b]).wait()

                @pl.when(jg + 1 < NGRP_A)
                def _(b=b):
                    pltpu.async_copy(y_hbm.at[sib[nxt].at[b]],
                                     rows[nxt][b], gsem[nxt][b])

            @pl.when(jg + 2 < NGRP_A)
            def _(jg=jg, cur=cur):
                pltpu.async_copy(tile_src.at[pl.ds((jg + 2) * G, G)],
                                 sib[cur], isem[cur])

            @pl.when(jg + 1 < NGRP_A)
            def _(jg=jg, nxt=nxt):
                pltpu.async_copy(tile_dst.at[pl.ds((jg + 1) * G, G)],
                                 dib[nxt], idsem[nxt])
        return carry

    lax.fori_loop(0, NGRP_A // 2, body, 0)
    for b in range(G):
        pltpu.make_async_copy(rows[1][b], acc_sh.at[dib[0].at[0]],
                              ssem[1][b]).wait()
    plsc.subcore_barrier()
    pltpu.sync_copy(acc_sh.at[pl.ds(s * RPT, RPT)],
                    out_hbm.at[pl.ds(c * N_PAD + s * RPT, RPT)])


_sc_agg = pl.kernel(
    _sc_agg_body,
    out_type=jax.ShapeDtypeStruct((NC * N_PAD, HD), jnp.float32),
    mesh=plsc.VectorSubcoreMesh(core_axis_name="c", subcore_axis_name="s"),
    scratch_types=[
        pltpu.VMEM((G, CH), jnp.int32),
        pltpu.VMEM((G, CH), jnp.int32),
        pltpu.VMEM((G, CH), jnp.int32),
        pltpu.VMEM((G, CH), jnp.int32),
        pltpu.VMEM((CH, HD), jnp.float32),
        pltpu.VMEM((CH, HD), jnp.float32),
        pltpu.VMEM((CH, HD), jnp.float32),
        pltpu.VMEM((CH, HD), jnp.float32),
        pltpu.VMEM((CH, HD), jnp.float32),
        pltpu.VMEM((CH, HD), jnp.float32),
        pltpu.VMEM((CH, HD), jnp.float32),
        pltpu.VMEM((CH, HD), jnp.float32),
        pltpu.VMEM_SHARED((N_PAD, HD), jnp.float32),
        pltpu.SemaphoreType.DMA,
        pltpu.SemaphoreType.DMA,
        pltpu.SemaphoreType.DMA,
        pltpu.SemaphoreType.DMA,
        pltpu.SemaphoreType.DMA,
        pltpu.SemaphoreType.DMA,
        pltpu.SemaphoreType.DMA,
        pltpu.SemaphoreType.DMA,
        pltpu.SemaphoreType.DMA,
        pltpu.SemaphoreType.DMA,
        pltpu.SemaphoreType.DMA,
        pltpu.SemaphoreType.DMA,
        pltpu.SemaphoreType.DMA,
        pltpu.SemaphoreType.DMA,
        pltpu.SemaphoreType.DMA,
        pltpu.SemaphoreType.DMA,
        pltpu.SemaphoreType.DMA,
        pltpu.SemaphoreType.DMA,
        pltpu.SemaphoreType.DMA,
        pltpu.SemaphoreType.DMA,
    ],
    compiler_params=pltpu.CompilerParams(use_tc_tiling_on_sc=False),
)


def _dis_block(d0_ref, d1_ref):
    return lax.rsqrt(d0_ref[...][:, 0:1] + d1_ref[...][:, 0:1] + 1.0)


def _tc_l1_body(x_ref, w_ref, d0_ref, d1_ref, y_ref):
    dis = _dis_block(d0_ref, d1_ref)
    y_ref[...] = dis * jnp.dot(x_ref[...], w_ref[...],
                               preferred_element_type=jnp.float32)


def _tc_mid_body(alo_ref, ahi_ref, yp_ref, d0_ref, d1_ref, b_ref, w_ref,
                 y_ref):
    dis = _dis_block(d0_ref, d1_ref)
    agg = jnp.concatenate([alo_ref[...], ahi_ref[...]], axis=1)
    pre = dis * (agg + yp_ref[...]) + b_ref[...]
    h = jax.nn.sigmoid(pre)
    y_ref[...] = dis * jnp.dot(h, w_ref[...],
                               preferred_element_type=jnp.float32)


def _tc_fin_body(alo_ref, ahi_ref, yp_ref, d0_ref, d1_ref, b_ref, out_ref):
    dis = _dis_block(d0_ref, d1_ref)
    agg = jnp.concatenate([alo_ref[...], ahi_ref[...]], axis=1)
    out_ref[...] = dis * (agg + yp_ref[...]) + b_ref[...]


_row_spec = pl.BlockSpec((BM, D), lambda m: (m, 0))
_alo_spec = pl.BlockSpec((BM, HD), lambda m: (m, 0))
_ahi_spec = pl.BlockSpec((BM, HD), lambda m: (GRID_M + m, 0))
_deg_spec = pl.BlockSpec((BM, 16), lambda m: (m, 0))
_w_spec = pl.BlockSpec((D, D), lambda m: (0, 0))
_b_spec = pl.BlockSpec((1, D), lambda m: (0, 0))
_out_sds = jax.ShapeDtypeStruct((N_PAD, D), jnp.float32)

_tc_l1 = pl.pallas_call(
    _tc_l1_body,
    grid=(GRID_M,),
    in_specs=[_row_spec, _w_spec, _deg_spec, _deg_spec],
    out_specs=_row_spec,
    out_shape=_out_sds,
)

_tc_mid = pl.pallas_call(
    _tc_mid_body,
    grid=(GRID_M,),
    in_specs=[_alo_spec, _ahi_spec, _row_spec, _deg_spec, _deg_spec,
              _b_spec, _w_spec],
    out_specs=_row_spec,
    out_shape=_out_sds,
)

_tc_fin = pl.pallas_call(
    _tc_fin_body,
    grid=(GRID_M,),
    in_specs=[_alo_spec, _ahi_spec, _row_spec, _deg_spec, _deg_spec, _b_spec],
    out_specs=_row_spec,
    out_shape=_out_sds,
)


def kernel(x, edge_index, edge_attr, W1, b1, W2, b2, W3, b3):
    src = edge_index[0].astype(jnp.int32)
    dst = edge_index[1].astype(jnp.int32)
    pad = E_PAD - src.shape[0]
    ar = jnp.arange(pad, dtype=jnp.int32)
    src_f = jnp.concatenate([src, ar % N])
    dst_f = jnp.concatenate([dst, N + (ar % (N_PAD - N))])
    dst_deg = dst_f.reshape(NW, NCHUNK_D, CH)
    src2 = src_f * 2
    srcx = jnp.stack([src2, src2 + 1]).reshape(NC, NS, NCHUNK_A, CH)
    dstx = dst_f.reshape(NS, NCHUNK_A, CH)

    ones16 = jnp.ones((CH, 16), jnp.float32)
    zeros16 = jnp.zeros((N_PAD, 16), jnp.float32)
    zerosH = jnp.zeros((N_PAD, HD), jnp.float32)
    x_pad = jnp.concatenate(
        [x, jnp.zeros((N_PAD - N, D), jnp.float32)], axis=0)

    degp = _sc_deg(dst_deg, ones16, zeros16)
    deg0 = degp[:N_PAD]
    deg1 = degp[N_PAD:]

    b1r = b1.reshape(1, D)
    b2r = b2.reshape(1, D)
    b3r = b3.reshape(1, D)

    y1 = _tc_l1(x_pad, W1, deg0, deg1)
    acc = _sc_agg(y1.reshape(NC * N_PAD, HD), srcx, dstx, zerosH)
    y2 = _tc_mid(acc, acc, y1, deg0, deg1, b1r, W2)
    acc = _sc_agg(y2.reshape(NC * N_PAD, HD), srcx, dstx, zerosH)
    y3 = _tc_mid(acc, acc, y2, deg0, deg1, b2r, W3)
    acc = _sc_agg(y3.reshape(NC * N_PAD, HD), srcx, dstx, zerosH)
    out = _tc_fin(acc, acc, y3, deg0, deg1, b3r)
    return out[:N]
